# baseline (speedup 1.0000x reference)
import functools

import jax
import jax.numpy as jnp
from jax.experimental import pallas as pl


def _matmul_kern(x_ref, w_ref, b_ref, o_ref):
    o_ref[...] = (
        jnp.dot(x_ref[...], w_ref[...], preferred_element_type=jnp.float32)
        + b_ref[...]
    )


def _pallas_matmul(x, w, b, block_rows=1000):
    n, k = x.shape
    ko, m = w.shape
    grid = (n // block_rows,)
    return pl.pallas_call(
        _matmul_kern,
        grid=grid,
        in_specs=[
            pl.BlockSpec((block_rows, k), lambda i: (i, 0)),
            pl.BlockSpec((k, m), lambda i: (0, 0)),
            pl.BlockSpec((m,), lambda i: (0,)),
        ],
        out_specs=pl.BlockSpec((block_rows, m), lambda i: (i, 0)),
        out_shape=jax.ShapeDtypeStruct((n, m), jnp.float32),
    )(x, w, b)


def _gat_layer(h, e_w, src, dst, Wm, b, a, snorm_n, att_ew):
    n, d = h.shape
    z = _pallas_matmul(h, Wm, b)
    a_s = a[:d, 0]
    a_d = a[d : 2 * d, 0]
    if att_ew:
        a_e_sum = jnp.sum(a[2 * d :, 0])
    ss = z @ a_s
    sd = z @ a_d
    logits = ss[src] + sd[dst]
    if att_ew:
        logits = logits + a_e_sum * e_w[:, 0]
    logits = jax.nn.leaky_relu(logits, 0.2)
    m = jax.ops.segment_max(logits, dst, num_segments=n)
    m = jnp.where(jnp.isfinite(m), m, 0.0)
    ex = jnp.exp(logits - m[dst])
    den = jax.ops.segment_sum(ex, dst, num_segments=n)
    alpha = ex / (den[dst] + 1e-9)
    agg = jax.ops.segment_sum(alpha[:, None] * z[src], dst, num_segments=n)
    return jax.nn.relu((h + agg) * snorm_n)


def _gat_vae(h, e_w, src, dst, p, pre, att_ew, snorm_n):
    h = _gat_layer(h, e_w, src, dst, p[pre + "1_W"], p[pre + "1_b"], p[pre + "1_a"], snorm_n, att_ew)
    h = _gat_layer(h, e_w, src, dst, p[pre + "2_W"], p[pre + "2_b"], p[pre + "2_a"], snorm_n, att_ew)
    return h


def kernel(feats, e_w, snorm_n, gt, maps_emb, params, edge_index):
    src = edge_index[0]
    dst = edge_index[1]
    p = params
    h_emb = _pallas_matmul(feats, p["emb_W"], p["emb_b"])
    h = jnp.concatenate([maps_emb, h_emb, gt], axis=-1)
    h = _gat_vae(h, e_w, src, dst, p, "enc", True, snorm_n)
    he = jnp.concatenate([h, gt], axis=-1)
    he = jax.nn.leaky_relu(_pallas_matmul(he, p["encl_W"], p["encl_b"]), 0.01)
    mu = _pallas_matmul(he, p["encmu_W"], p["encmu_b"])
    log_var = _pallas_matmul(he, p["enclv_W"], p["enclv_b"])
    hp = jnp.concatenate([maps_emb, h_emb], axis=-1)
    hp = _gat_vae(hp, e_w, src, dst, p, "pri", True, snorm_n)
    hp2 = jax.nn.leaky_relu(_pallas_matmul(hp, p["pril_W"], p["pril_b"]), 0.01)
    mu_p = _pallas_matmul(hp2, p["primu_W"], p["primu_b"])
    log_var_p = _pallas_matmul(hp2, p["prilv_W"], p["prilv_b"])
    eps = jax.random.normal(jax.random.key(42), mu.shape, dtype=jnp.float32)
    z = mu + jnp.exp(0.5 * log_var) * eps
    hd = jnp.concatenate([h_emb, z], axis=-1)
    hd = _gat_vae(hd, e_w, src, dst, p, "dec", False, snorm_n)
    recon = _pallas_matmul(jnp.concatenate([hd, z], axis=-1), p["out_W"], p["out_b"])
    return (recon, mu, log_var, mu_p, log_var_p)
